# Initial kernel scaffold; baseline (speedup 1.0000x reference)
import jax
import jax.numpy as jnp
from jax.experimental import pallas as pl


def kernel(patches, table):
    raise NotImplementedError("write your pallas kernel here")


# speedup vs baseline: 1.5795x; 1.5795x over previous
import jax
import jax.numpy as jnp
from jax.experimental import pallas as pl


def _body(p_ref, t_ref, o_ref):
    D = p_ref.shape[-1]
    o_ref[:, :, :D] = p_ref[...]
    o_ref[:, :, D:] = jnp.broadcast_to(t_ref[...][None], o_ref[:, :, D:].shape)


def kernel(patches, table):
    B, P, D = patches.shape
    E = table.shape[1]
    return pl.pallas_call(
        _body,
        grid=(B,),
        in_specs=[
            pl.BlockSpec((1, P, D), lambda b: (b, 0, 0)),
            pl.BlockSpec((P, E), lambda b: (0, 0)),
        ],
        out_specs=pl.BlockSpec((1, P, D + E), lambda b: (b, 0, 0)),
        out_shape=jax.ShapeDtypeStruct((B, P, D + E), patches.dtype),
    )(patches, table)


# speedup vs baseline: 1.7074x; 1.0810x over previous
import jax
import jax.numpy as jnp
from jax.experimental import pallas as pl


def _body(p_ref, t_ref, o_ref):
    D = p_ref.shape[-1]
    o_ref[:, :, :D] = p_ref[...]
    o_ref[:, :, D:] = jnp.broadcast_to(t_ref[...][None], o_ref[:, :, D:].shape)


def kernel(patches, table):
    B, P, D = patches.shape
    E = table.shape[1]
    BB = 4
    return pl.pallas_call(
        _body,
        grid=(B // BB,),
        in_specs=[
            pl.BlockSpec((BB, P, D), lambda b: (b, 0, 0)),
            pl.BlockSpec((P, E), lambda b: (0, 0)),
        ],
        out_specs=pl.BlockSpec((BB, P, D + E), lambda b: (b, 0, 0)),
        out_shape=jax.ShapeDtypeStruct((B, P, D + E), patches.dtype),
    )(patches, table)
